# baseline (speedup 1.0000x reference)
import functools

import jax
import jax.numpy as jnp
from jax import lax
from jax.experimental import pallas as pl
from jax.experimental.pallas import tpu as pltpu
from jax.experimental.pallas import tpu_sc as plsc

NC = 2
NS = 16
NW = NC * NS
LANES = 16
GPC = 32
CROWS = GPC * LANES
NBUF = 2


@functools.partial(jax.jit, static_argnames=("n_rows", "d"))
def _sc_lookup(idx_flat, table_flat, *, n_rows, d):
    rows_per_w = n_rows // NW
    n_chunks = rows_per_w // CROWS
    cwords = CROWS * d
    assert n_chunks % 2 == 0 and n_chunks >= 4

    mesh = plsc.VectorSubcoreMesh(core_axis_name="c", subcore_axis_name="s")

    @functools.partial(
        pl.kernel,
        out_type=jax.ShapeDtypeStruct((n_rows * d,), jnp.float32),
        mesh=mesh,
        scratch_types=[
            pltpu.VMEM((table_flat.shape[0],), jnp.float32),
            pltpu.VMEM((rows_per_w,), jnp.int32),
            tuple(pltpu.VMEM((cwords,), jnp.float32) for _ in range(NBUF)),
            tuple(pltpu.SemaphoreType.DMA for _ in range(NBUF)),
            pltpu.SemaphoreType.DMA,
        ],
        compiler_params=pltpu.CompilerParams(
            use_tc_tiling_on_sc=False, needs_layout_passes=False),
    )
    def body(table_hbm, idx_hbm, out_hbm, table_v, idx_v, bufs, sw, sem0):
        wid = lax.axis_index("s") * NC + lax.axis_index("c")
        ibase = wid * rows_per_w
        wbase = wid * rows_per_w * d

        pltpu.async_copy(table_hbm, table_v, sem0).wait()
        pltpu.async_copy(idx_hbm.at[pl.ds(ibase, rows_per_w)], idx_v, sem0).wait()

        lanev = lax.broadcasted_iota(jnp.int32, (LANES,), 0) * d

        def fill(chunk, b):
            def grp(j, carry):
                r0 = chunk * CROWS + j * LANES
                iv = idx_v[pl.ds(r0, LANES)]
                ivx = iv * d
                pvec = lanev + j * (LANES * d)
                for c in range(d):
                    vals = plsc.load_gather(table_v, [ivx + c])
                    plsc.store_scatter(bufs[b], [pvec + c], vals)
                return carry

            lax.fori_loop(0, GPC, grp, 0)

        def start_w(chunk, b):
            pltpu.async_copy(
                bufs[b], out_hbm.at[pl.ds(wbase + chunk * cwords, cwords)],
                sw[b])

        def wait_w(chunk, b):
            pltpu.make_async_copy(
                bufs[b], out_hbm.at[pl.ds(wbase + chunk * cwords, cwords)],
                sw[b]).wait()

        fill(0, 0)
        start_w(0, 0)
        fill(1, 1)
        start_w(1, 1)

        def pair_body(p, carry):
            for k in range(2):
                c = 2 * p + 2 + k
                b = k
                wait_w(c - 2, b)
                fill(c, b)
                start_w(c, b)
            return carry

        lax.fori_loop(0, (n_chunks - 2) // 2, pair_body, 0)

        wait_w(n_chunks - 2, 0)
        wait_w(n_chunks - 1, 1)

    return body(table_flat, idx_flat)


def kernel(hop_distances, embedding_weight):
    b, t = hop_distances.shape
    _, d = embedding_weight.shape
    n_rows = b * t
    idx_flat = hop_distances.astype(jnp.int32).reshape(-1)
    table_flat = embedding_weight[1:].reshape(-1)
    out = _sc_lookup(idx_flat, table_flat, n_rows=n_rows, d=d)
    return out.reshape(b, t, d)


# speedup vs baseline: 3.9426x; 3.9426x over previous
import functools

import jax
import jax.numpy as jnp
from jax import lax
from jax.experimental import pallas as pl
from jax.experimental.pallas import tpu as pltpu
from jax.experimental.pallas import tpu_sc as plsc

NC = 2
NS = 16
NW = NC * NS
CHUNK = 128
GK = 4
NBUF = 3


@functools.partial(jax.jit, static_argnames=("n_rows", "d"))
def _sc_lookup(idx_grouped, table, *, n_rows, d):
    rows_per_w = n_rows // NW
    n_chunks = rows_per_w // CHUNK
    n_blocks = n_chunks // GK
    n_emb = table.shape[0]
    assert n_blocks >= NBUF

    mesh = plsc.VectorSubcoreMesh(core_axis_name="c", subcore_axis_name="s")

    @functools.partial(
        pl.kernel,
        out_type=jax.ShapeDtypeStruct((n_rows, d), jnp.float32),
        mesh=mesh,
        scratch_types=[
            pltpu.VMEM_SHARED((NS * (n_emb - 1), d), jnp.float32),
            pltpu.VMEM((rows_per_w,), jnp.int32),
            tuple(pltpu.VMEM((GK * CHUNK, d), jnp.float32) for _ in range(NBUF)),
            tuple(pltpu.SemaphoreType.DMA for _ in range(NBUF)),
            tuple(pltpu.SemaphoreType.DMA for _ in range(NBUF)),
            pltpu.SemaphoreType.DMA,
        ],
        compiler_params=pltpu.CompilerParams(use_tc_tiling_on_sc=False),
    )
    def body(table_hbm, idx_hbm, out_hbm, table_sh, idx_v, rows, sg, sw, sem0):
        wid = lax.axis_index("s") * NC + lax.axis_index("c")
        base = wid * rows_per_w
        blk = GK * CHUNK

        sid = lax.axis_index("s")
        pltpu.async_copy(
            table_hbm.at[pl.ds(1, n_emb - 1)],
            table_sh.at[pl.ds(sid * (n_emb - 1), n_emb - 1)], sem0).wait()
        pltpu.async_copy(idx_hbm.at[pl.ds(base, rows_per_w)], idx_v, sem0).wait()

        roff = sid * (n_emb - 1)

        def off_body(k, carry):
            sl = pl.ds(k * 16, 16)
            idx_v[sl] = idx_v[sl] + roff
            return carry

        lax.fori_loop(0, rows_per_w // 16, off_body, 0)

        def start_g(i, b):
            pltpu.async_copy(
                table_sh.at[idx_v.at[pl.ds(i * blk, blk)]], rows[b], sg[b])

        def wait_g(i, b):
            pltpu.make_async_copy(
                table_sh.at[idx_v.at[pl.ds(i * blk, blk)]], rows[b], sg[b]).wait()

        def start_w(i, b):
            pltpu.async_copy(
                rows[b], out_hbm.at[pl.ds(base + i * blk, blk)], sw[b])

        def wait_w(i, b):
            pltpu.make_async_copy(
                rows[b], out_hbm.at[pl.ds(base + i * blk, blk)], sw[b]).wait()

        start_g(0, 0)
        start_g(1, 1)
        for i in range(n_blocks):
            b = i % NBUF
            wait_g(i, b)
            if i >= 1:
                wait_w(i - 1, (i - 1) % NBUF)
            if i + 2 < n_blocks:
                start_g(i + 2, (i + 2) % NBUF)
            start_w(i, b)
        wait_w(n_blocks - 1, (n_blocks - 1) % NBUF)

    return body(table, idx_grouped)


def kernel(hop_distances, embedding_weight):
    b, t = hop_distances.shape
    _, d = embedding_weight.shape
    n_rows = b * t
    idx_grouped = hop_distances.astype(jnp.int32).reshape(-1)
    out = _sc_lookup(idx_grouped, embedding_weight, n_rows=n_rows, d=d)
    return out.reshape(b, t, d)
